# baseline (speedup 1.0000x reference)
import jax
import jax.numpy as jnp
from jax.experimental import pallas as pl
from jax.experimental.pallas import tpu as pltpu

_POS_DIM = 16
_HID_DIM = 256
_TB = 4096
_NSPLIT = 16


def _half(x, pos, w_refs, b_refs):
    w1, w2, w3, w4a, w4b, w5 = w_refs
    b1, b2, b3, b4, b5 = b_refs
    h = jnp.dot(x, w1, preferred_element_type=jnp.float32) + b1
    h = jnp.maximum(h, 0.0)
    h = jnp.dot(h, w2, preferred_element_type=jnp.float32) + b2
    h = jnp.maximum(h, 0.0)
    lf = jnp.dot(h, w3, preferred_element_type=jnp.float32) + b3
    t = (jnp.dot(lf, w4a, preferred_element_type=jnp.float32)
         + jnp.dot(pos, w4b, preferred_element_type=jnp.float32)
         + b4)
    p = jnp.tanh(t)
    return jnp.tanh(jnp.dot(p, w5, preferred_element_type=jnp.float32) + b5)


def _mlp_kernel(*refs):
    x_refs = refs[:_NSPLIT]
    p_refs = refs[_NSPLIT:2 * _NSPLIT]
    (w1_ref, w2_ref, w3_ref, w4a_ref, w4b_ref, w5_ref,
     b1_ref, b2_ref, b3_ref, b4_ref, b5_ref, out_ref) = refs[2 * _NSPLIT:]
    w_refs = (w1_ref[0], w2_ref[0], w3_ref[0], w4a_ref[0],
              w4b_ref[0][:_POS_DIM, :], w5_ref[0])
    b_refs = (b1_ref[0], b2_ref[0], b3_ref[0], b4_ref[0], b5_ref[0])
    th = x_refs[0].shape[0]
    for i in range(_NSPLIT):
        out_ref[i * th:(i + 1) * th, :] = _half(
            x_refs[i][...], p_refs[i][...], w_refs, b_refs)


def kernel(lidar_state, position_state, w_slab, b_slab):
    B, L = lidar_state.shape
    H, POSD, NS = _HID_DIM, _POS_DIM, _NSPLIT

    TB = min(_TB, B)
    assert B % TB == 0 and TB % NS == 0
    TH = TB // NS

    def xspec(i):
        return pl.BlockSpec((TH, L), lambda b, i=i: (NS * b + i, 0))

    def pspec(i):
        return pl.BlockSpec((TH, POSD), lambda b, i=i: (NS * b + i, 0))

    def wspec(layer, rows, cols, row_block=0):
        return pl.BlockSpec((1, rows, cols),
                            lambda b, la=layer, rb=row_block: (la, rb, 0))

    def bspec(layer, cols):
        return pl.BlockSpec((1, 1, cols), lambda b, la=layer: (la, 0, 0))

    out = pl.pallas_call(
        _mlp_kernel,
        out_shape=jax.ShapeDtypeStruct((B, H), jnp.float32),
        grid=(B // TB,),
        in_specs=(
            [xspec(i) for i in range(NS)]
            + [pspec(i) for i in range(NS)]
            + [
                wspec(0, L, H),
                wspec(1, H, H),
                wspec(2, H, 128),
                wspec(3, 128, H),
                wspec(3, 72, H, 15),
                wspec(4, H, H),
                bspec(0, H), bspec(1, H), bspec(2, 128),
                bspec(3, H), bspec(4, H),
            ]
        ),
        out_specs=pl.BlockSpec((TB, H), lambda b: (b, 0)),
        compiler_params=pltpu.CompilerParams(
            dimension_semantics=("parallel",)),
    )(*([lidar_state] * NS), *([position_state] * NS),
      w_slab, w_slab, w_slab, w_slab, w_slab, w_slab,
      b_slab, b_slab, b_slab, b_slab, b_slab)
    return out


# speedup vs baseline: 1.0415x; 1.0415x over previous
import jax
import jax.numpy as jnp
from jax.experimental import pallas as pl
from jax.experimental.pallas import tpu as pltpu

_POS_DIM = 16
_HID_DIM = 256
_TB = 4096
_NSPLIT = 8


def _half(x, pos, w_refs, b_refs):
    w1, w2, w3, w4a, w4b, w5 = w_refs
    b1, b2, b3, b4, b5 = b_refs
    h = jnp.dot(x, w1, preferred_element_type=jnp.float32) + b1
    h = jnp.maximum(h, 0.0)
    h = jnp.dot(h, w2, preferred_element_type=jnp.float32) + b2
    h = jnp.maximum(h, 0.0)
    lf = jnp.dot(h, w3, preferred_element_type=jnp.float32) + b3
    t = (jnp.dot(lf, w4a, preferred_element_type=jnp.float32)
         + jnp.dot(pos, w4b, preferred_element_type=jnp.float32)
         + b4)
    p = jnp.tanh(t)
    return jnp.tanh(jnp.dot(p, w5, preferred_element_type=jnp.float32) + b5)


def _mlp_kernel(*refs):
    x_refs = refs[:_NSPLIT]
    p_refs = refs[_NSPLIT:2 * _NSPLIT]
    (w1_ref, w2_ref, w3_ref, w4a_ref, w4b_ref, w5_ref,
     b1_ref, b2_ref, b3_ref, b4_ref, b5_ref, out_ref) = refs[2 * _NSPLIT:]
    w_refs = (w1_ref[0], w2_ref[0], w3_ref[0], w4a_ref[0],
              w4b_ref[0][:_POS_DIM, :], w5_ref[0])
    b_refs = (b1_ref[0], b2_ref[0], b3_ref[0], b4_ref[0], b5_ref[0])
    th = x_refs[0].shape[0] * x_refs[0].shape[1]
    for i in range(_NSPLIT):
        x = x_refs[i][...].reshape(th, x_refs[i].shape[2])
        out_ref[i * th:(i + 1) * th, :] = _half(
            x, p_refs[i][...], w_refs, b_refs)


def kernel(lidar_state, position_state, w_slab, b_slab):
    B, L = lidar_state.shape
    H, POSD, NS = _HID_DIM, _POS_DIM, _NSPLIT

    TB = min(_TB, B)
    assert B % TB == 0 and TB % NS == 0
    TH = TB // NS

    x3 = lidar_state.reshape(B // 8, 8, L)

    def xspec(i):
        return pl.BlockSpec((TH // 8, 8, L), lambda b, i=i: (NS * b + i, 0, 0))

    def pspec(i):
        return pl.BlockSpec((TH, POSD), lambda b, i=i: (NS * b + i, 0))

    def wspec(layer, rows, cols, row_block=0):
        return pl.BlockSpec((1, rows, cols),
                            lambda b, la=layer, rb=row_block: (la, rb, 0))

    def bspec(layer, cols):
        return pl.BlockSpec((1, 1, cols), lambda b, la=layer: (la, 0, 0))

    out = pl.pallas_call(
        _mlp_kernel,
        out_shape=jax.ShapeDtypeStruct((B, H), jnp.float32),
        grid=(B // TB,),
        in_specs=(
            [xspec(i) for i in range(NS)]
            + [pspec(i) for i in range(NS)]
            + [
                wspec(0, L, H),
                wspec(1, H, H),
                wspec(2, H, 128),
                wspec(3, 128, H),
                wspec(3, 72, H, 15),
                wspec(4, H, H),
                bspec(0, H), bspec(1, H), bspec(2, 128),
                bspec(3, H), bspec(4, H),
            ]
        ),
        out_specs=pl.BlockSpec((TB, H), lambda b: (b, 0)),
        compiler_params=pltpu.CompilerParams(
            dimension_semantics=("parallel",)),
    )(*([x3] * NS), *([position_state] * NS),
      w_slab, w_slab, w_slab, w_slab, w_slab, w_slab,
      b_slab, b_slab, b_slab, b_slab, b_slab)
    return out


# speedup vs baseline: 1.0492x; 1.0075x over previous
import jax
import jax.numpy as jnp
from jax.experimental import pallas as pl
from jax.experimental.pallas import tpu as pltpu

_POS_DIM = 16
_HID_DIM = 256
_TB = 4096
_NSPLIT = 8


def _half(x, pos, w_refs, b_refs):
    w1, w2, w3, w4a, w4b, w5 = w_refs
    b1, b2, b3, b4, b5 = b_refs
    h = jnp.dot(x, w1, preferred_element_type=jnp.float32) + b1
    h = jnp.maximum(h, 0.0)
    h = jnp.dot(h, w2, preferred_element_type=jnp.float32) + b2
    h = jnp.maximum(h, 0.0)
    lf = jnp.dot(h, w3, preferred_element_type=jnp.float32) + b3
    t = (jnp.dot(lf, w4a, preferred_element_type=jnp.float32)
         + jnp.dot(pos, w4b, preferred_element_type=jnp.float32)
         + b4)
    p = jnp.tanh(t)
    return jnp.tanh(jnp.dot(p, w5, preferred_element_type=jnp.float32) + b5)


def _mlp_kernel(*refs):
    x_refs = refs[:_NSPLIT]
    p_refs = refs[_NSPLIT:2 * _NSPLIT]
    (w1_ref, w2_ref, w3_ref, w4a_ref, w4b_ref, w5_ref,
     b1_ref, b2_ref, b3_ref, b4_ref, b5_ref, out_ref) = refs[2 * _NSPLIT:]
    w_refs = (w1_ref[0], w2_ref[0], w3_ref[0], w4a_ref[0],
              w4b_ref[0][:_POS_DIM, :], w5_ref[0])
    b_refs = (b1_ref[0], b2_ref[0], b3_ref[0], b4_ref[0], b5_ref[0])
    th = x_refs[0].shape[0]
    for i in range(_NSPLIT):
        out_ref[i * th:(i + 1) * th, :] = _half(
            x_refs[i][...], p_refs[i][...], w_refs, b_refs)


def kernel(lidar_state, position_state, w_slab, b_slab):
    B, L = lidar_state.shape
    H, POSD, NS = _HID_DIM, _POS_DIM, _NSPLIT

    TB = min(_TB, B)
    assert B % TB == 0 and TB % NS == 0
    TH = TB // NS

    lidar_b = jax.lax.optimization_barrier(lidar_state)

    def xspec(i):
        return pl.BlockSpec((TH, L), lambda b, i=i: (NS * b + i, 0))

    def pspec(i):
        return pl.BlockSpec((TH, POSD), lambda b, i=i: (NS * b + i, 0))

    def wspec(layer, rows, cols, row_block=0):
        return pl.BlockSpec((1, rows, cols),
                            lambda b, la=layer, rb=row_block: (la, rb, 0))

    def bspec(layer, cols):
        return pl.BlockSpec((1, 1, cols), lambda b, la=layer: (la, 0, 0))

    out = pl.pallas_call(
        _mlp_kernel,
        out_shape=jax.ShapeDtypeStruct((B, H), jnp.float32),
        grid=(B // TB,),
        in_specs=(
            [xspec(i) for i in range(NS)]
            + [pspec(i) for i in range(NS)]
            + [
                wspec(0, L, H),
                wspec(1, H, H),
                wspec(2, H, 128),
                wspec(3, 128, H),
                wspec(3, 72, H, 15),
                wspec(4, H, H),
                bspec(0, H), bspec(1, H), bspec(2, 128),
                bspec(3, H), bspec(4, H),
            ]
        ),
        out_specs=pl.BlockSpec((TB, H), lambda b: (b, 0)),
        compiler_params=pltpu.CompilerParams(
            dimension_semantics=("parallel",)),
    )(*([lidar_b] * NS), *([position_state] * NS),
      w_slab, w_slab, w_slab, w_slab, w_slab, w_slab,
      b_slab, b_slab, b_slab, b_slab, b_slab)
    return out


# speedup vs baseline: 1.0531x; 1.0037x over previous
import jax
import jax.numpy as jnp
from jax.experimental import pallas as pl
from jax.experimental.pallas import tpu as pltpu

_POS_DIM = 16
_HID_DIM = 256
_TB = 4096
_NSPLIT = 8


def _half(x, pos, w_refs, b_refs):
    w1, w2, w3, w4a, w4b, w5 = w_refs
    b1, b2, b3, b4, b5 = b_refs
    h = jnp.dot(x, w1, preferred_element_type=jnp.float32) + b1
    h = jnp.maximum(h, 0.0)
    h = jnp.dot(h, w2, preferred_element_type=jnp.float32) + b2
    h = jnp.maximum(h, 0.0)
    lf = jnp.dot(h, w3, preferred_element_type=jnp.float32) + b3
    t = (jnp.dot(lf, w4a, preferred_element_type=jnp.float32)
         + jnp.dot(pos, w4b, preferred_element_type=jnp.float32)
         + b4)
    p = jnp.tanh(t)
    return jnp.tanh(jnp.dot(p, w5, preferred_element_type=jnp.float32) + b5)


def _mlp_kernel(*refs):
    x_refs = refs[:_NSPLIT]
    p_refs = refs[_NSPLIT:2 * _NSPLIT]
    (w1_ref, w2_ref, w3_ref, w4a_ref, w4b_ref, w5_ref,
     b1_ref, b2_ref, b3_ref, b4_ref, b5_ref, out_ref) = refs[2 * _NSPLIT:]
    w_refs = (w1_ref[0], w2_ref[0], w3_ref[0], w4a_ref[0],
              w4b_ref[0][:_POS_DIM, :], w5_ref[0])
    b_refs = (b1_ref[0], b2_ref[0], b3_ref[0], b4_ref[0], b5_ref[0])
    th = x_refs[0].shape[0]
    for i in range(_NSPLIT):
        out_ref[i * th:(i + 1) * th, :] = _half(
            x_refs[i][...], p_refs[i][...], w_refs, b_refs)


def kernel(lidar_state, position_state, w_slab, b_slab):
    B, L = lidar_state.shape
    H, POSD, NS = _HID_DIM, _POS_DIM, _NSPLIT

    TB = min(_TB, B)
    assert B % TB == 0 and TB % NS == 0
    TH = TB // NS

    def xspec(i):
        return pl.BlockSpec((TH, L), lambda b, i=i: (NS * b + i, 0))

    def pspec(i):
        return pl.BlockSpec((TH, POSD), lambda b, i=i: (NS * b + i, 0))

    def wspec(layer, rows, cols, row_block=0):
        return pl.BlockSpec((1, rows, cols),
                            lambda b, la=layer, rb=row_block: (la, rb, 0))

    def bspec(layer, cols):
        return pl.BlockSpec((1, 1, cols), lambda b, la=layer: (la, 0, 0))

    out = pl.pallas_call(
        _mlp_kernel,
        out_shape=jax.ShapeDtypeStruct((B, H), jnp.float32),
        grid=(B // TB,),
        in_specs=(
            [xspec(i) for i in range(NS)]
            + [pspec(i) for i in range(NS)]
            + [
                wspec(0, L, H),
                wspec(1, H, H),
                wspec(2, H, 128),
                wspec(3, 128, H),
                wspec(3, 72, H, 15),
                wspec(4, H, H),
                bspec(0, H), bspec(1, H), bspec(2, 128),
                bspec(3, H), bspec(4, H),
            ]
        ),
        out_specs=pl.BlockSpec((TB, H), lambda b: (b, 0)),
        compiler_params=pltpu.CompilerParams(
            dimension_semantics=("parallel",)),
    )(*([lidar_state] * NS), *([position_state] * NS),
      w_slab, w_slab, w_slab, w_slab, w_slab, w_slab,
      b_slab, b_slab, b_slab, b_slab, b_slab)
    return out
